# baseline (speedup 1.0000x reference)
import functools

import jax
import jax.numpy as jnp
from jax import lax
from jax.experimental import pallas as pl
from jax.experimental.pallas import tpu as pltpu
from jax.experimental.pallas import tpu_sc as plsc

BATCH = 4096
SEQ = 200
EMBED = 64
VOCAB = 1000000

NUM_CORES = 2
NUM_SUBCORES = 16
NUM_WORKERS = NUM_CORES * NUM_SUBCORES
BPW = BATCH // NUM_WORKERS
SBLK = SEQ // 8


@functools.partial(
    pl.kernel,
    out_type=jax.ShapeDtypeStruct((SEQ, EMBED // 8, NUM_WORKERS, 8 * BPW), jnp.float32),
    mesh=plsc.VectorSubcoreMesh(core_axis_name="c", subcore_axis_name="s"),
    compiler_params=pltpu.CompilerParams(
        use_tc_tiling_on_sc=False, needs_layout_passes=False),
    scratch_types=[
        pltpu.VMEM((SBLK, 8, BPW), jnp.int32),
        pltpu.VMEM((2, BPW, EMBED), jnp.float32),
        pltpu.VMEM((2, 8, 8 * BPW), jnp.float32),
        pltpu.SemaphoreType.DMA,
        pltpu.SemaphoreType.DMA,
        pltpu.SemaphoreType.DMA,
        pltpu.SemaphoreType.DMA,
    ],
)
def _sc_embed(x_hbm, tok_hbm, pimg_hbm, out_hbm,
              idx_v, rows_v, til_v, g0, g1, o0, o1):
    wid = lax.axis_index("s") * NUM_CORES + lax.axis_index("c")

    pltpu.sync_copy(x_hbm.at[:, wid], idx_v)

    def scale_flat(i, carry):
        blk = i // 64
        row = (i // 8) % 8
        col = (i % 8) * 16
        v = idx_v[blk, row, pl.ds(col, 16)]
        idx_v[blk, row, pl.ds(col, 16)] = v + v
        return carry
    lax.fori_loop(0, SBLK * 8 * 8, scale_flat, 0)

    def gather(s, p):
        gsem = (g0, g1)[p]
        return pltpu.async_copy(
            tok_hbm.at[idx_v.at[s // 8, s % 8]], rows_v.at[p], gsem)

    def wait_gather(p):
        gsem = (g0, g1)[p]
        pltpu.make_async_copy(
            tok_hbm.at[idx_v.at[0, 0]], rows_v.at[p], gsem).wait()

    def store(s, p):
        osem = (o0, o1)[p]
        return pltpu.async_copy(til_v.at[p], out_hbm.at[s, :, wid], osem)

    def wait_store(s, p):
        osem = (o0, o1)[p]
        pltpu.make_async_copy(til_v.at[p], out_hbm.at[s, :, wid], osem).wait()

    gather(0, 0)

    def s_body(step, carry):
        for p in range(2):
            s = step * 2 + p

            @pl.when(s + 1 < SEQ)
            def _():
                gather(s + 1, 1 - p)

            @pl.when(s >= 2)
            def _():
                wait_store(s, p)

            pltpu.sync_copy(pimg_hbm.at[s], til_v.at[p])

            wait_gather(p)

            iota = lax.iota(jnp.int32, 16)
            tvecs = [iota + 16 * tc for tc in range(8)]
            for et in range(8):
                for el in range(8):
                    e = jnp.full((16,), et * 8 + el, jnp.int32)
                    for tc in range(8):
                        val = plsc.load_gather(
                            rows_v.at[p], [tvecs[tc], e])
                        plsc.addupdate(
                            til_v.at[p, et, pl.ds(el * BPW + tc * 16, 16)],
                            val)

            store(s, p)
        return carry

    lax.fori_loop(0, SEQ // 2, s_body, 0)
    wait_store(SEQ - 2, 0)
    wait_store(SEQ - 1, 1)


def kernel(x, token_table, position_table):
    xq = (
        jnp.transpose(x.astype(jnp.int32))
        .reshape(SBLK, 8, NUM_WORKERS, BPW)
        .transpose(0, 2, 1, 3)
    )
    tp = jnp.pad(token_table, ((0, 0), (0, 128 - EMBED))).reshape(2 * VOCAB, EMBED)
    pimg = jnp.broadcast_to(
        position_table.reshape(SEQ, 8, 8, 1), (SEQ, 8, 8, BPW)
    ).reshape(SEQ, 8, 8 * BPW)
    out6 = _sc_embed(xq, tp, pimg)
    return (
        out6.reshape(SEQ, 8, NUM_WORKERS, 8, BPW)
        .transpose(2, 4, 0, 1, 3)
        .reshape(BATCH, SEQ, EMBED)
    )


# speedup vs baseline: 1.1584x; 1.1584x over previous
import functools

import jax
import jax.numpy as jnp
from jax import lax
from jax.experimental import pallas as pl
from jax.experimental.pallas import tpu as pltpu
from jax.experimental.pallas import tpu_sc as plsc

BATCH = 4096
SEQ = 200
EMBED = 64
VOCAB = 1000000

NUM_CORES = 2
NUM_SUBCORES = 16
NUM_WORKERS = NUM_CORES * NUM_SUBCORES
BPW = BATCH // NUM_WORKERS
SBLK = SEQ // 8


@functools.partial(
    pl.kernel,
    out_type=jax.ShapeDtypeStruct((SEQ, EMBED // 8, NUM_WORKERS, 8 * BPW), jnp.float32),
    mesh=plsc.VectorSubcoreMesh(core_axis_name="c", subcore_axis_name="s"),
    compiler_params=pltpu.CompilerParams(
        use_tc_tiling_on_sc=False, needs_layout_passes=False),
    scratch_types=[
        pltpu.VMEM((SBLK, 8, BPW), jnp.int32),
        pltpu.VMEM((2, BPW, EMBED), jnp.float32),
        pltpu.VMEM((2, 8, 8 * BPW), jnp.float32),
        pltpu.VMEM((2, 8, 8 * BPW), jnp.float32),
        pltpu.SemaphoreType.DMA,
        pltpu.SemaphoreType.DMA,
        pltpu.SemaphoreType.DMA,
        pltpu.SemaphoreType.DMA,
        pltpu.SemaphoreType.DMA,
        pltpu.SemaphoreType.DMA,
    ],
)
def _sc_embed(x_hbm, tok_hbm, pimg_hbm, out_hbm,
              idx_v, rows_v, til_v, ptl_v, g0, g1, o0, o1, q0, q1):
    wid = lax.axis_index("s") * NUM_CORES + lax.axis_index("c")

    pltpu.sync_copy(x_hbm.at[:, wid], idx_v)

    def scale_flat(i, carry):
        blk = i // 64
        row = (i // 8) % 8
        col = (i % 8) * 16
        v = idx_v[blk, row, pl.ds(col, 16)]
        idx_v[blk, row, pl.ds(col, 16)] = v + v
        return carry
    lax.fori_loop(0, SBLK * 8 * 8, scale_flat, 0)

    def gather(s, p):
        gsem = (g0, g1)[p]
        return pltpu.async_copy(
            tok_hbm.at[idx_v.at[s // 8, s % 8]], rows_v.at[p], gsem)

    def wait_gather(p):
        gsem = (g0, g1)[p]
        pltpu.make_async_copy(
            tok_hbm.at[idx_v.at[0, 0]], rows_v.at[p], gsem).wait()

    def prefill(s, p):
        qsem = (q0, q1)[p]
        return pltpu.async_copy(pimg_hbm.at[s], ptl_v.at[p], qsem)

    def wait_prefill(s, p):
        qsem = (q0, q1)[p]
        pltpu.make_async_copy(pimg_hbm.at[s], ptl_v.at[p], qsem).wait()

    def store(s, p):
        osem = (o0, o1)[p]
        return pltpu.async_copy(til_v.at[p], out_hbm.at[s, :, wid], osem)

    def wait_store(s, p):
        osem = (o0, o1)[p]
        pltpu.make_async_copy(til_v.at[p], out_hbm.at[s, :, wid], osem).wait()

    gather(0, 0)
    prefill(0, 0)

    def s_body(step, carry):
        for p in range(2):
            s = step * 2 + p

            @pl.when(s + 1 < SEQ)
            def _():
                gather(s + 1, 1 - p)
                prefill(s + 1, 1 - p)

            @pl.when(s >= 2)
            def _():
                wait_store(s, p)

            wait_gather(p)
            wait_prefill(s, p)

            iota = lax.iota(jnp.int32, 16)
            tvecs = [iota + 16 * tc for tc in range(8)]

            def et_body(et, ecarry):
                for el in range(8):
                    e = jnp.full((16,), et * 8 + el, jnp.int32)
                    for tc in range(8):
                        sl = pl.ds(el * BPW + tc * 16, 16)
                        val = plsc.load_gather(rows_v.at[p], [tvecs[tc], e])
                        til_v[p, et, sl] = val + ptl_v[p, et, sl]
                return ecarry
            lax.fori_loop(0, 8, et_body, 0)

            store(s, p)
        return carry

    lax.fori_loop(0, SEQ // 2, s_body, 0)
    wait_store(SEQ - 2, 0)
    wait_store(SEQ - 1, 1)


def kernel(x, token_table, position_table):
    xq = (
        jnp.transpose(x.astype(jnp.int32))
        .reshape(SBLK, 8, NUM_WORKERS, BPW)
        .transpose(0, 2, 1, 3)
    )
    tp = jnp.pad(token_table, ((0, 0), (0, 128 - EMBED))).reshape(2 * VOCAB, EMBED)
    pimg = jnp.broadcast_to(
        position_table.reshape(SEQ, 8, 8, 1), (SEQ, 8, 8, BPW)
    ).reshape(SEQ, 8, 8 * BPW)
    out6 = _sc_embed(xq, tp, pimg)
    return (
        out6.reshape(SEQ, 8, NUM_WORKERS, 8, BPW)
        .transpose(2, 4, 0, 1, 3)
        .reshape(BATCH, SEQ, EMBED)
    )


# speedup vs baseline: 1.6945x; 1.4628x over previous
import functools

import jax
import jax.numpy as jnp
from jax import lax
from jax.experimental import pallas as pl
from jax.experimental.pallas import tpu as pltpu
from jax.experimental.pallas import tpu_sc as plsc

BATCH = 4096
SEQ = 200
EMBED = 64
VOCAB = 1000000
HALF = SEQ // 2

NUM_CORES = 2
NUM_SUBCORES = 16
NUM_WORKERS = NUM_CORES * NUM_SUBCORES
BPW = BATCH // NUM_WORKERS
G = 2
NGRP = BPW // G

_CHUNKS = (0, 16, 32, 48, 64, 80, 84)


@functools.partial(
    pl.kernel,
    out_type=jax.ShapeDtypeStruct((BATCH, 2, HALF, EMBED), jnp.float32),
    mesh=plsc.VectorSubcoreMesh(core_axis_name="c", subcore_axis_name="s"),
    compiler_params=pltpu.CompilerParams(
        use_tc_tiling_on_sc=False, needs_layout_passes=False),
    scratch_types=[
        pltpu.VMEM((G, 2, HALF), jnp.int32),
        pltpu.VMEM((2, G, 2, HALF), jnp.int32),
        pltpu.VMEM((2, G, 2, HALF, EMBED), jnp.float32),
        pltpu.VMEM((2, HALF, EMBED), jnp.float32),
        pltpu.SemaphoreType.DMA,
        pltpu.SemaphoreType.DMA,
        pltpu.SemaphoreType.DMA,
        pltpu.SemaphoreType.DMA,
    ],
)
def _sc_embed(x_hbm, tok_hbm, pos_hbm, out_hbm,
              raw_v, idx_v, rows_v, pos_v, g0, g1, o0, o1):
    wid = lax.axis_index("s") * NUM_CORES + lax.axis_index("c")
    base = wid * BPW

    pltpu.sync_copy(pos_hbm, pos_v)

    def stage_and_fire(j, p):
        b0 = base + j * G
        pltpu.sync_copy(x_hbm.at[pl.ds(b0, G)], raw_v)
        for g in range(G):
            for h in range(2):
                for c in _CHUNKS:
                    v = raw_v[g, h, pl.ds(c, 16)]
                    idx_v[p, g, h, pl.ds(c, 16)] = v + v
        gsem = (g0, g1)[p]
        for g in range(G):
            for h in range(2):
                pltpu.async_copy(
                    tok_hbm.at[idx_v.at[p, g, h]], rows_v.at[p, g, h], gsem)

    def wait_gathers(p):
        gsem = (g0, g1)[p]
        for _ in range(G * 2):
            pltpu.make_async_copy(
                tok_hbm.at[idx_v.at[0, 0, 0]], rows_v.at[p, 0, 0], gsem
            ).wait()

    def store(j, p):
        osem = (o0, o1)[p]
        return pltpu.async_copy(
            rows_v.at[p], out_hbm.at[pl.ds(base + j * G, G)], osem)

    def wait_store(j, p):
        osem = (o0, o1)[p]
        pltpu.make_async_copy(
            rows_v.at[p], out_hbm.at[pl.ds(base + j * G, G)], osem).wait()

    stage_and_fire(0, 0)

    def j_body(step, carry):
        for p in range(2):
            j = step * 2 + p

            wait_gathers(p)

            @pl.when(j + 1 < NGRP)
            def _():
                @pl.when(j >= 1)
                def _():
                    wait_store(j - 1, 1 - p)
                stage_and_fire(j + 1, 1 - p)

            def r_body(r, rcarry):
                for h in range(2):
                    prow = [pos_v[h, r, pl.ds(16 * c, 16)] for c in range(4)]
                    for g in range(G):
                        for c in range(4):
                            sl = pl.ds(16 * c, 16)
                            rows_v[p, g, h, r, sl] = (
                                rows_v[p, g, h, r, sl] + prow[c])
                return rcarry
            lax.fori_loop(0, HALF, r_body, 0)

            store(j, p)
        return carry

    lax.fori_loop(0, NGRP // 2, j_body, 0)
    wait_store(NGRP - 2, 0)
    wait_store(NGRP - 1, 1)


def kernel(x, token_table, position_table):
    x2 = x.astype(jnp.int32).reshape(BATCH, 2, HALF)
    tp = jnp.pad(token_table, ((0, 0), (0, 128 - EMBED))).reshape(2 * VOCAB, EMBED)
    pos2 = position_table.reshape(2, HALF, EMBED)
    out = _sc_embed(x2, tp, pos2)
    return out.reshape(BATCH, SEQ, EMBED)


# speedup vs baseline: 1.7440x; 1.0292x over previous
import functools

import jax
import jax.numpy as jnp
from jax import lax
from jax.experimental import pallas as pl
from jax.experimental.pallas import tpu as pltpu
from jax.experimental.pallas import tpu_sc as plsc

BATCH = 4096
SEQ = 200
EMBED = 64
VOCAB = 1000000
HALF = SEQ // 2

NUM_CORES = 2
NUM_SUBCORES = 16
NUM_WORKERS = NUM_CORES * NUM_SUBCORES
BPW = BATCH // NUM_WORKERS
G = 4
NGRP = BPW // G

_CHUNKS = (0, 16, 32, 48, 64, 80, 84)


@functools.partial(
    pl.kernel,
    out_type=jax.ShapeDtypeStruct((BATCH, 2, HALF, EMBED), jnp.float32),
    mesh=plsc.VectorSubcoreMesh(core_axis_name="c", subcore_axis_name="s"),
    compiler_params=pltpu.CompilerParams(
        use_tc_tiling_on_sc=False, needs_layout_passes=False),
    scratch_types=[
        pltpu.VMEM((G, 2, HALF), jnp.int32),
        pltpu.VMEM((2, G, 2, HALF), jnp.int32),
        pltpu.VMEM((2, G, 2, HALF, EMBED), jnp.float32),
        pltpu.VMEM((2, HALF, EMBED), jnp.float32),
        pltpu.SemaphoreType.DMA,
        pltpu.SemaphoreType.DMA,
        pltpu.SemaphoreType.DMA,
        pltpu.SemaphoreType.DMA,
    ],
)
def _sc_embed(x_hbm, tok_hbm, pos_hbm, out_hbm,
              raw_v, idx_v, rows_v, pos_v, g0, g1, o0, o1):
    wid = lax.axis_index("s") * NUM_CORES + lax.axis_index("c")
    base = wid * BPW

    pltpu.sync_copy(pos_hbm, pos_v)

    def stage_and_fire(j, p):
        b0 = base + j * G
        pltpu.sync_copy(x_hbm.at[pl.ds(b0, G)], raw_v)
        for g in range(G):
            for h in range(2):
                for c in _CHUNKS:
                    v = raw_v[g, h, pl.ds(c, 16)]
                    idx_v[p, g, h, pl.ds(c, 16)] = v + v
        gsem = (g0, g1)[p]
        for g in range(G):
            for h in range(2):
                pltpu.async_copy(
                    tok_hbm.at[idx_v.at[p, g, h]], rows_v.at[p, g, h], gsem)

    def wait_gathers(p):
        gsem = (g0, g1)[p]
        for _ in range(G * 2):
            pltpu.make_async_copy(
                tok_hbm.at[idx_v.at[0, 0, 0]], rows_v.at[p, 0, 0], gsem
            ).wait()

    def store(j, p):
        osem = (o0, o1)[p]
        return pltpu.async_copy(
            rows_v.at[p], out_hbm.at[pl.ds(base + j * G, G)], osem)

    def wait_store(j, p):
        osem = (o0, o1)[p]
        pltpu.make_async_copy(
            rows_v.at[p], out_hbm.at[pl.ds(base + j * G, G)], osem).wait()

    stage_and_fire(0, 0)

    def j_body(step, carry):
        for p in range(2):
            j = step * 2 + p

            wait_gathers(p)

            @pl.when(j + 1 < NGRP)
            def _():
                @pl.when(j >= 1)
                def _():
                    wait_store(j - 1, 1 - p)
                stage_and_fire(j + 1, 1 - p)

            def r_body(r, rcarry):
                for h in range(2):
                    prow = [pos_v[h, r, pl.ds(16 * c, 16)] for c in range(4)]
                    for g in range(G):
                        for c in range(4):
                            sl = pl.ds(16 * c, 16)
                            rows_v[p, g, h, r, sl] = (
                                rows_v[p, g, h, r, sl] + prow[c])
                return rcarry
            lax.fori_loop(0, HALF, r_body, 0)

            store(j, p)
        return carry

    lax.fori_loop(0, NGRP // 2, j_body, 0)
    wait_store(NGRP - 2, 0)
    wait_store(NGRP - 1, 1)


def kernel(x, token_table, position_table):
    x2 = x.astype(jnp.int32).reshape(BATCH, 2, HALF)
    tp = jnp.pad(token_table, ((0, 0), (0, 128 - EMBED))).reshape(2 * VOCAB, EMBED)
    pos2 = position_table.reshape(2, HALF, EMBED)
    out = _sc_embed(x2, tp, pos2)
    return out.reshape(BATCH, SEQ, EMBED)
